# baseline (speedup 1.0000x reference)
import functools

import jax
import jax.numpy as jnp
from jax import lax
from jax.experimental import pallas as pl
from jax.experimental.pallas import tpu as pltpu
from jax.experimental.pallas import tpu_sc as plsc

D = 189
CH = 128
L = 16
K = 13
NBUF = 3


@functools.lru_cache(maxsize=None)
def _build_sc_call(B):
    info = plsc.get_sparse_core_info()
    NC, NS = info.num_cores, info.num_subcores
    NW = NC * NS
    BPW = B // NW
    NCHUNK = BPW // CH
    NJOB = 2 * NCHUNK
    GRP = CH // L

    mesh = plsc.VectorSubcoreMesh(core_axis_name="c", subcore_axis_name="s")

    @functools.partial(
        pl.kernel,
        out_type=jax.ShapeDtypeStruct((B,), jnp.float32),
        mesh=mesh,
        compiler_params=pltpu.CompilerParams(
            needs_layout_passes=False, use_tc_tiling_on_sc=False),
        scratch_types=[
            pltpu.VMEM((NCHUNK, K, CH), jnp.int32),
            pltpu.VMEM((NCHUNK, K, CH), jnp.int32),
            pltpu.VMEM((NCHUNK, CH), jnp.int32),
            pltpu.VMEM((NCHUNK, CH), jnp.int32),
            pltpu.VMEM((2 * D * L,), jnp.float32),
            pltpu.VMEM((L,), jnp.float32),
            pltpu.VMEM((K, CH, L), jnp.float32),
            pltpu.VMEM((K, CH, L), jnp.float32),
            pltpu.VMEM((K, CH, L), jnp.float32),
            pltpu.VMEM((BPW,), jnp.float32),
            pltpu.SemaphoreType.DMA,
            pltpu.SemaphoreType.DMA,
            pltpu.SemaphoreType.DMA,
        ],
    )
    def sc_kernel(e0v, e1v, gu, gi, ou, oi, ab_hbm, c_hbm, out_hbm,
                  gu_v, gi_v, ou_v, oi_v, a_v, c_v, buf0, buf1, buf2, acc_v,
                  s0, s1, s2):
        wid = lax.axis_index("s") * NC + lax.axis_index("c")
        base = wid * BPW
        crow = wid * NCHUNK

        pltpu.sync_copy(gu.at[pl.ds(crow, NCHUNK)], gu_v)
        pltpu.sync_copy(gi.at[pl.ds(crow, NCHUNK)], gi_v)
        pltpu.sync_copy(ou.at[pl.ds(crow, NCHUNK)], ou_v)
        pltpu.sync_copy(oi.at[pl.ds(crow, NCHUNK)], oi_v)
        pltpu.sync_copy(ab_hbm, a_v)
        pltpu.sync_copy(c_hbm, c_v)
        cvec = c_v[...]

        bufs = (buf0, buf1, buf2)
        sems = (s0, s1, s2)
        lane = lax.iota(jnp.int32, L)
        rvecs = [jnp.int32(g * L) + lane for g in range(GRP)]
        zero = jnp.zeros((L,), jnp.float32)

        def fire(j, buf, sem):
            tab = e0v if j < NCHUNK else e1v
            gv = gu_v if j < NCHUNK else gi_v
            c = j % NCHUNK
            return [pltpu.async_copy(tab.at[gv.at[c, kk]], buf.at[kk], sem)
                    for kk in range(K)]

        def compute_chunk(buf, t, c, coff):
            ov = ou_v if t == 0 else oi_v
            abase = t * D * L
            ovecs = [ov[c, pl.ds(g * L, L)] for g in range(GRP)]

            def dbody(d, accs):
                ab = a_v[pl.ds(abase + d * L, L)]
                new = []
                for g in range(GRP):
                    pos = ovecs[g] + d
                    kv = pos >> 4
                    lv = pos & 15
                    col = plsc.load_gather(buf, [kv, rvecs[g], lv])
                    new.append(accs[g] + col * ab)
                return tuple(new)

            accs = lax.fori_loop(0, D, dbody, (zero,) * GRP)
            for g in range(GRP):
                sl = pl.ds(coff + g * L, L)
                if t == 0:
                    acc_v[sl] = accs[g]
                else:
                    x = acc_v[sl] + accs[g] + cvec
                    acc_v[sl] = 1.0 / (1.0 + jnp.exp(-x))

        copies = [None] * NJOB
        for j in range(min(NBUF, NJOB)):
            copies[j] = fire(j, bufs[j % NBUF], sems[j % NBUF])

        for j in range(NJOB):
            buf = bufs[j % NBUF]
            for cp in copies[j]:
                cp.wait()
            t = 0 if j < NCHUNK else 1
            compute_chunk(buf, t, j % NCHUNK, (j % NCHUNK) * CH)
            if j + NBUF < NJOB:
                copies[j + NBUF] = fire(j + NBUF, buf, sems[j % NBUF])

        pltpu.sync_copy(acc_v, out_hbm.at[pl.ds(base, BPW)])

    return sc_kernel


def _prep_indices(idx, V):
    w = idx.astype(jnp.int32) * D
    s = w // L
    o = w % L
    nsub = (V * D) // L
    g = jnp.minimum(s[:, None] + jnp.arange(K, dtype=jnp.int32)[None, :],
                    nsub - 1)
    g = g.reshape(-1, CH, K).transpose(0, 2, 1)
    o = o.reshape(-1, CH)
    return g, o


def kernel(sparse_feature, E0, E1, Wu1, bu1, Wu2, bu2, Wi1, bi1, Wi2, bi2,
           W3, b3):
    B = sparse_feature.shape[0]
    V0 = E0.shape[0]
    V1 = E1.shape[0]
    user_idx = sparse_feature[:, 0].astype(jnp.int32)
    item_idx = sparse_feature[:, 1].astype(jnp.int32)

    a0 = (Wu1 @ Wu2 @ W3[:10]).reshape(-1)
    a1 = (Wi1 @ Wi2 @ W3[10:]).reshape(-1)
    c = ((bu1 @ Wu2 + bu2) @ W3[:10, 0]
         + (bi1 @ Wi2 + bi2) @ W3[10:, 0] + b3[0])

    ab = jnp.stack([a0, a1]).astype(jnp.float32)
    ab = jnp.broadcast_to(ab[:, :, None], (2, D, L)).reshape(-1)
    cv = jnp.full((L,), c, jnp.float32)

    gu, ou = _prep_indices(user_idx, V0)
    gi, oi = _prep_indices(item_idx, V1)

    e0v = E0.reshape(-1, L)
    e1v = E1.reshape(-1, L)

    sc_call = _build_sc_call(B)
    return sc_call(e0v, e1v, gu, gi, ou, oi, ab, cv)


# speedup vs baseline: 3.5046x; 3.5046x over previous
import functools

import jax
import jax.numpy as jnp
from jax import lax
from jax.experimental import pallas as pl
from jax.experimental.pallas import tpu as pltpu
from jax.experimental.pallas import tpu_sc as plsc

D = 189
MC = 128
TD = D - MC
CH = 128
L = 16
NBUF = 3
BLK = 8000


def _tail_body(e_ref, a_ref, o_ref):
    x = e_ref[...]
    col = lax.broadcasted_iota(jnp.int32, x.shape, 1)
    x = jnp.where(col < TD, x, 0.0)
    o_ref[...] = x @ a_ref[...]


@functools.lru_cache(maxsize=None)
def _build_tail_call(V):
    return pl.pallas_call(
        _tail_body,
        grid=(V // BLK,),
        in_specs=[
            pl.BlockSpec((BLK, MC), lambda i: (i, 1)),
            pl.BlockSpec((MC, 1), lambda i: (0, 0)),
        ],
        out_specs=pl.BlockSpec((BLK, 1), lambda i: (i, 0)),
        out_shape=jax.ShapeDtypeStruct((V, 1), jnp.float32),
    )


def _tail_dots(E, a):
    V = E.shape[0]
    tails = _build_tail_call(V)(E, a[MC:D, None].astype(jnp.float32))
    nsub = (V + MC - 1) // MC
    flat = jnp.pad(tails.reshape(-1), (0, nsub * MC - V))
    return flat.reshape(nsub, MC)


@functools.lru_cache(maxsize=None)
def _build_sc_call(B):
    info = plsc.get_sparse_core_info()
    NC, NS = info.num_cores, info.num_subcores
    NW = NC * NS
    BPW = B // NW
    NCHUNK = BPW // CH
    NJOB = 2 * NCHUNK
    GRP = CH // L

    mesh = plsc.VectorSubcoreMesh(core_axis_name="c", subcore_axis_name="s")

    @functools.partial(
        pl.kernel,
        out_type=jax.ShapeDtypeStruct((B,), jnp.float32),
        mesh=mesh,
        compiler_params=pltpu.CompilerParams(
            needs_layout_passes=False, use_tc_tiling_on_sc=True),
        scratch_types=[
            pltpu.VMEM((NCHUNK, CH), jnp.int32),
            pltpu.VMEM((NCHUNK, CH), jnp.int32),
            pltpu.VMEM((NCHUNK, CH), jnp.int32),
            pltpu.VMEM((NCHUNK, CH), jnp.int32),
            pltpu.VMEM((NCHUNK, CH), jnp.int32),
            pltpu.VMEM((NCHUNK, CH), jnp.int32),
            pltpu.VMEM((2 * MC * L,), jnp.float32),
            pltpu.VMEM((L,), jnp.float32),
            pltpu.VMEM((CH, MC), jnp.float32),
            pltpu.VMEM((CH, MC), jnp.float32),
            pltpu.VMEM((CH, MC), jnp.float32),
            pltpu.VMEM((CH, MC), jnp.float32),
            pltpu.VMEM((CH, MC), jnp.float32),
            pltpu.VMEM((CH, MC), jnp.float32),
            pltpu.VMEM((BPW,), jnp.float32),
            pltpu.SemaphoreType.DMA,
            pltpu.SemaphoreType.DMA,
            pltpu.SemaphoreType.DMA,
        ],
    )
    def sc_kernel(e0, e1, t0, t1, iu, ii, su, si, lu, li, ab_hbm, c_hbm,
                  out_hbm, iu_v, ii_v, su_v, si_v, lu_v, li_v, a_v, c_v,
                  ma0, ma1, ma2, ta0, ta1, ta2, acc_v, s0, s1, s2):
        wid = lax.axis_index("s") * NC + lax.axis_index("c")
        base = wid * BPW
        crow = wid * NCHUNK

        pltpu.sync_copy(iu.at[pl.ds(crow, NCHUNK)], iu_v)
        pltpu.sync_copy(ii.at[pl.ds(crow, NCHUNK)], ii_v)
        pltpu.sync_copy(su.at[pl.ds(crow, NCHUNK)], su_v)
        pltpu.sync_copy(si.at[pl.ds(crow, NCHUNK)], si_v)
        pltpu.sync_copy(lu.at[pl.ds(crow, NCHUNK)], lu_v)
        pltpu.sync_copy(li.at[pl.ds(crow, NCHUNK)], li_v)
        pltpu.sync_copy(ab_hbm, a_v)
        pltpu.sync_copy(c_hbm, c_v)
        cvec = c_v[...]

        mbufs = (ma0, ma1, ma2)
        tbufs = (ta0, ta1, ta2)
        sems = (s0, s1, s2)
        lane = lax.iota(jnp.int32, L)
        rvecs = [jnp.int32(g * L) + lane for g in range(GRP)]
        zero = jnp.zeros((L,), jnp.float32)

        def fire(j, mbuf, tbuf, sem):
            tab, tls = (e0, t0) if j < NCHUNK else (e1, t1)
            iv = iu_v if j < NCHUNK else ii_v
            sv = su_v if j < NCHUNK else si_v
            c = j % NCHUNK
            return [
                pltpu.async_copy(tab.at[iv.at[c], pl.ds(0, MC)], mbuf, sem),
                pltpu.async_copy(tls.at[sv.at[c]], tbuf, sem),
            ]

        def compute_chunk(mbuf, tbuf, t, c, coff):
            lv = lu_v if t == 0 else li_v
            abase = t * MC * L

            def dbody(d, accs):
                ab = a_v[pl.ds(abase + d * L, L)]
                dvec = jnp.full((L,), d, jnp.int32)
                return tuple(
                    accs[g] + plsc.load_gather(mbuf, [rvecs[g], dvec]) * ab
                    for g in range(GRP))

            accs = lax.fori_loop(0, MC, dbody, (zero,) * GRP)
            for g in range(GRP):
                tl = lv[c, pl.ds(g * L, L)]
                tv = plsc.load_gather(tbuf, [rvecs[g], tl])
                sl = pl.ds(coff + g * L, L)
                if t == 0:
                    acc_v[sl] = accs[g] + tv
                else:
                    x = acc_v[sl] + accs[g] + tv + cvec
                    acc_v[sl] = 1.0 / (1.0 + jnp.exp(-x))

        copies = [None] * NJOB
        for j in range(min(NBUF, NJOB)):
            copies[j] = fire(j, mbufs[j % NBUF], tbufs[j % NBUF],
                             sems[j % NBUF])

        for j in range(NJOB):
            for cp in copies[j]:
                cp.wait()
            t = 0 if j < NCHUNK else 1
            compute_chunk(mbufs[j % NBUF], tbufs[j % NBUF], t,
                          j % NCHUNK, (j % NCHUNK) * CH)
            if j + NBUF < NJOB:
                copies[j + NBUF] = fire(j + NBUF, mbufs[j % NBUF],
                                        tbufs[j % NBUF], sems[j % NBUF])

        pltpu.sync_copy(acc_v, out_hbm.at[pl.ds(base, BPW)])

    return sc_kernel


def kernel(sparse_feature, E0, E1, Wu1, bu1, Wu2, bu2, Wi1, bi1, Wi2, bi2,
           W3, b3):
    B = sparse_feature.shape[0]
    user_idx = sparse_feature[:, 0].astype(jnp.int32)
    item_idx = sparse_feature[:, 1].astype(jnp.int32)

    a0 = (Wu1 @ Wu2 @ W3[:10]).reshape(-1)
    a1 = (Wi1 @ Wi2 @ W3[10:]).reshape(-1)
    c = ((bu1 @ Wu2 + bu2) @ W3[:10, 0]
         + (bi1 @ Wi2 + bi2) @ W3[10:, 0] + b3[0])

    ab = jnp.stack([a0[:MC], a1[:MC]]).astype(jnp.float32)
    ab = jnp.broadcast_to(ab[:, :, None], (2, MC, L)).reshape(-1)
    cv = jnp.full((L,), c, jnp.float32)

    t0 = _tail_dots(E0, a0)
    t1 = _tail_dots(E1, a1)

    iu = user_idx.reshape(-1, CH)
    ii = item_idx.reshape(-1, CH)
    su = iu >> 7
    si = ii >> 7
    lu = iu & (MC - 1)
    li = ii & (MC - 1)

    sc_call = _build_sc_call(B)
    return sc_call(E0, E1, t0, t1, iu, ii, su, si, lu, li, ab, cv)


# speedup vs baseline: 3.5190x; 1.0041x over previous
import functools

import jax
import jax.numpy as jnp
from jax import lax
from jax.experimental import pallas as pl
from jax.experimental.pallas import tpu as pltpu
from jax.experimental.pallas import tpu_sc as plsc

D = 189
MC = 128
TD = D - MC
CH = 128
L = 16
NBUF = 3
BLK = 20000


def _tail_body(e_ref, a_ref, o_ref):
    y = e_ref[...] @ a_ref[...]
    o_ref[...] = y[:, :1]


@functools.lru_cache(maxsize=None)
def _build_tail_call(V):
    return pl.pallas_call(
        _tail_body,
        grid=(V // BLK,),
        in_specs=[
            pl.BlockSpec((BLK, MC), lambda i: (i, 1)),
            pl.BlockSpec((MC, MC), lambda i: (0, 0)),
        ],
        out_specs=pl.BlockSpec((BLK, 1), lambda i: (i, 0)),
        out_shape=jax.ShapeDtypeStruct((V, 1), jnp.float32),
    )


def _tail_dots(E, a):
    V = E.shape[0]
    a_mat = jnp.zeros((MC, MC), jnp.float32).at[:TD, 0].set(
        a[MC:D].astype(jnp.float32))
    tails = _build_tail_call(V)(E, a_mat)
    nsub = (V + MC - 1) // MC
    flat = jnp.pad(tails.reshape(-1), (0, nsub * MC - V))
    return flat.reshape(nsub, MC)


@functools.lru_cache(maxsize=None)
def _build_sc_call(B):
    info = plsc.get_sparse_core_info()
    NC, NS = info.num_cores, info.num_subcores
    NW = NC * NS
    BPW = B // NW
    NCHUNK = BPW // CH
    NJOB = 2 * NCHUNK
    GRP = CH // L

    mesh = plsc.VectorSubcoreMesh(core_axis_name="c", subcore_axis_name="s")

    @functools.partial(
        pl.kernel,
        out_type=jax.ShapeDtypeStruct((B,), jnp.float32),
        mesh=mesh,
        compiler_params=pltpu.CompilerParams(
            needs_layout_passes=False, use_tc_tiling_on_sc=True),
        scratch_types=[
            pltpu.VMEM((NCHUNK, CH), jnp.int32),
            pltpu.VMEM((NCHUNK, CH), jnp.int32),
            pltpu.VMEM((NCHUNK, CH), jnp.int32),
            pltpu.VMEM((NCHUNK, CH), jnp.int32),
            pltpu.VMEM((NCHUNK, CH), jnp.int32),
            pltpu.VMEM((NCHUNK, CH), jnp.int32),
            pltpu.VMEM((2 * MC * L,), jnp.float32),
            pltpu.VMEM((L,), jnp.float32),
            pltpu.VMEM((CH, MC), jnp.float32),
            pltpu.VMEM((CH, MC), jnp.float32),
            pltpu.VMEM((CH, MC), jnp.float32),
            pltpu.VMEM((CH, MC), jnp.float32),
            pltpu.VMEM((CH, MC), jnp.float32),
            pltpu.VMEM((CH, MC), jnp.float32),
            pltpu.VMEM((BPW,), jnp.float32),
            pltpu.SemaphoreType.DMA,
            pltpu.SemaphoreType.DMA,
            pltpu.SemaphoreType.DMA,
        ],
    )
    def sc_kernel(e0, e1, t0, t1, iu, ii, su, si, lu, li, ab_hbm, c_hbm,
                  out_hbm, iu_v, ii_v, su_v, si_v, lu_v, li_v, a_v, c_v,
                  ma0, ma1, ma2, ta0, ta1, ta2, acc_v, s0, s1, s2):
        wid = lax.axis_index("s") * NC + lax.axis_index("c")
        base = wid * BPW
        crow = wid * NCHUNK

        pltpu.sync_copy(iu.at[pl.ds(crow, NCHUNK)], iu_v)
        pltpu.sync_copy(ii.at[pl.ds(crow, NCHUNK)], ii_v)
        pltpu.sync_copy(su.at[pl.ds(crow, NCHUNK)], su_v)
        pltpu.sync_copy(si.at[pl.ds(crow, NCHUNK)], si_v)
        pltpu.sync_copy(lu.at[pl.ds(crow, NCHUNK)], lu_v)
        pltpu.sync_copy(li.at[pl.ds(crow, NCHUNK)], li_v)
        pltpu.sync_copy(ab_hbm, a_v)
        pltpu.sync_copy(c_hbm, c_v)
        cvec = c_v[...]

        mbufs = (ma0, ma1, ma2)
        tbufs = (ta0, ta1, ta2)
        sems = (s0, s1, s2)
        lane = lax.iota(jnp.int32, L)
        rvecs = [jnp.int32(g * L) + lane for g in range(GRP)]
        zero = jnp.zeros((L,), jnp.float32)

        def fire(j, mbuf, tbuf, sem):
            tab, tls = (e0, t0) if j < NCHUNK else (e1, t1)
            iv = iu_v if j < NCHUNK else ii_v
            sv = su_v if j < NCHUNK else si_v
            c = j % NCHUNK
            return [
                pltpu.async_copy(tab.at[iv.at[c], pl.ds(0, MC)], mbuf, sem),
                pltpu.async_copy(tls.at[sv.at[c]], tbuf, sem),
            ]

        def compute_chunk(mbuf, tbuf, t, c, coff):
            lv = lu_v if t == 0 else li_v
            abase = t * MC * L

            def dbody(d, accs):
                ab = a_v[pl.ds(abase + d * L, L)]
                dvec = jnp.full((L,), d, jnp.int32)
                return tuple(
                    accs[g] + plsc.load_gather(mbuf, [rvecs[g], dvec]) * ab
                    for g in range(GRP))

            accs = lax.fori_loop(0, MC, dbody, (zero,) * GRP)
            for g in range(GRP):
                tl = lv[c, pl.ds(g * L, L)]
                tv = plsc.load_gather(tbuf, [rvecs[g], tl])
                sl = pl.ds(coff + g * L, L)
                if t == 0:
                    acc_v[sl] = accs[g] + tv
                else:
                    x = acc_v[sl] + accs[g] + tv + cvec
                    acc_v[sl] = 1.0 / (1.0 + jnp.exp(-x))

        copies = [None] * NJOB
        for j in range(min(NBUF, NJOB)):
            copies[j] = fire(j, mbufs[j % NBUF], tbufs[j % NBUF],
                             sems[j % NBUF])

        for j in range(NJOB):
            for cp in copies[j]:
                cp.wait()
            t = 0 if j < NCHUNK else 1
            compute_chunk(mbufs[j % NBUF], tbufs[j % NBUF], t,
                          j % NCHUNK, (j % NCHUNK) * CH)
            if j + NBUF < NJOB:
                copies[j + NBUF] = fire(j + NBUF, mbufs[j % NBUF],
                                        tbufs[j % NBUF], sems[j % NBUF])

        pltpu.sync_copy(acc_v, out_hbm.at[pl.ds(base, BPW)])

    return sc_kernel


def kernel(sparse_feature, E0, E1, Wu1, bu1, Wu2, bu2, Wi1, bi1, Wi2, bi2,
           W3, b3):
    B = sparse_feature.shape[0]
    user_idx = sparse_feature[:, 0].astype(jnp.int32)
    item_idx = sparse_feature[:, 1].astype(jnp.int32)

    a0 = (Wu1 @ Wu2 @ W3[:10]).reshape(-1)
    a1 = (Wi1 @ Wi2 @ W3[10:]).reshape(-1)
    c = ((bu1 @ Wu2 + bu2) @ W3[:10, 0]
         + (bi1 @ Wi2 + bi2) @ W3[10:, 0] + b3[0])

    ab = jnp.stack([a0[:MC], a1[:MC]]).astype(jnp.float32)
    ab = jnp.broadcast_to(ab[:, :, None], (2, MC, L)).reshape(-1)
    cv = jnp.full((L,), c, jnp.float32)

    t0 = _tail_dots(E0, a0)
    t1 = _tail_dots(E1, a1)

    iu = user_idx.reshape(-1, CH)
    ii = item_idx.reshape(-1, CH)
    su = iu >> 7
    si = ii >> 7
    lu = iu & (MC - 1)
    li = ii & (MC - 1)

    sc_call = _build_sc_call(B)
    return sc_call(E0, E1, t0, t1, iu, ii, su, si, lu, li, ab, cv)


# speedup vs baseline: 18.9616x; 5.3883x over previous
import functools

import jax
import jax.numpy as jnp
from jax import lax
from jax.experimental import pallas as pl
from jax.experimental.pallas import tpu as pltpu
from jax.experimental.pallas import tpu_sc as plsc

D = 189
MC = 128
CH = 128
L = 16
NBUF = 3
VBLK = 16384


def _dots_body(a_ref, et_ref, o_ref):
    o_ref[...] = (a_ref[...] @ et_ref[...])[0]


@functools.lru_cache(maxsize=None)
def _build_dots_call(V, VP):
    grid = (VP + VBLK - 1) // VBLK
    return pl.pallas_call(
        _dots_body,
        grid=(grid,),
        in_specs=[
            pl.BlockSpec((1, D), lambda i: (0, 0)),
            pl.BlockSpec((D, VBLK), lambda i: (0, i)),
        ],
        out_specs=pl.BlockSpec((VBLK,), lambda i: (i,)),
        out_shape=jax.ShapeDtypeStruct((VP,), jnp.float32),
    )


def _folded_dots(E, a):
    V = E.shape[0]
    nsub = (V + MC - 1) // MC
    VP = nsub * MC
    tv = _build_dots_call(V, VP)(a[None, :].astype(jnp.float32), E.T)
    return tv.reshape(nsub, MC)


@functools.lru_cache(maxsize=None)
def _build_sc_call(B):
    info = plsc.get_sparse_core_info()
    NC, NS = info.num_cores, info.num_subcores
    NW = NC * NS
    BPW = B // NW
    NCHUNK = BPW // CH
    GRP = CH // L

    mesh = plsc.VectorSubcoreMesh(core_axis_name="c", subcore_axis_name="s")

    @functools.partial(
        pl.kernel,
        out_type=jax.ShapeDtypeStruct((B,), jnp.float32),
        mesh=mesh,
        compiler_params=pltpu.CompilerParams(
            needs_layout_passes=False, use_tc_tiling_on_sc=True),
        scratch_types=[
            pltpu.VMEM((NCHUNK, CH), jnp.int32),
            pltpu.VMEM((NCHUNK, CH), jnp.int32),
            pltpu.VMEM((NCHUNK, CH), jnp.int32),
            pltpu.VMEM((NCHUNK, CH), jnp.int32),
            pltpu.VMEM((L,), jnp.float32),
            pltpu.VMEM((CH, MC), jnp.float32),
            pltpu.VMEM((CH, MC), jnp.float32),
            pltpu.VMEM((CH, MC), jnp.float32),
            pltpu.VMEM((CH, MC), jnp.float32),
            pltpu.VMEM((CH, MC), jnp.float32),
            pltpu.VMEM((CH, MC), jnp.float32),
            pltpu.VMEM((BPW,), jnp.float32),
            pltpu.SemaphoreType.DMA,
            pltpu.SemaphoreType.DMA,
            pltpu.SemaphoreType.DMA,
        ],
    )
    def sc_kernel(t0, t1, su, si, lu, li, c_hbm, out_hbm,
                  su_v, si_v, lu_v, li_v, c_v,
                  ua0, ua1, ua2, ia0, ia1, ia2, acc_v, s0, s1, s2):
        wid = lax.axis_index("s") * NC + lax.axis_index("c")
        base = wid * BPW
        crow = wid * NCHUNK

        pltpu.sync_copy(su.at[pl.ds(crow, NCHUNK)], su_v)
        pltpu.sync_copy(si.at[pl.ds(crow, NCHUNK)], si_v)
        pltpu.sync_copy(lu.at[pl.ds(crow, NCHUNK)], lu_v)
        pltpu.sync_copy(li.at[pl.ds(crow, NCHUNK)], li_v)
        pltpu.sync_copy(c_hbm, c_v)
        cvec = c_v[...]

        ubufs = (ua0, ua1, ua2)
        ibufs = (ia0, ia1, ia2)
        sems = (s0, s1, s2)
        lane = lax.iota(jnp.int32, L)
        rvecs = [jnp.int32(g * L) + lane for g in range(GRP)]

        def fire(c, ubuf, ibuf, sem):
            return [
                pltpu.async_copy(t0.at[su_v.at[c]], ubuf, sem),
                pltpu.async_copy(t1.at[si_v.at[c]], ibuf, sem),
            ]

        copies = [None] * NCHUNK
        for c in range(min(NBUF, NCHUNK)):
            copies[c] = fire(c, ubufs[c % NBUF], ibufs[c % NBUF],
                             sems[c % NBUF])

        for c in range(NCHUNK):
            for cp in copies[c]:
                cp.wait()
            ubuf, ibuf = ubufs[c % NBUF], ibufs[c % NBUF]
            for g in range(GRP):
                tlu = lu_v[c, pl.ds(g * L, L)]
                tli = li_v[c, pl.ds(g * L, L)]
                tu = plsc.load_gather(ubuf, [rvecs[g], tlu])
                ti = plsc.load_gather(ibuf, [rvecs[g], tli])
                x = tu + ti + cvec
                acc_v[pl.ds(c * CH + g * L, L)] = 1.0 / (1.0 + jnp.exp(-x))
            if c + NBUF < NCHUNK:
                copies[c + NBUF] = fire(c + NBUF, ubuf, ibuf, sems[c % NBUF])

        pltpu.sync_copy(acc_v, out_hbm.at[pl.ds(base, BPW)])

    return sc_kernel


def kernel(sparse_feature, E0, E1, Wu1, bu1, Wu2, bu2, Wi1, bi1, Wi2, bi2,
           W3, b3):
    B = sparse_feature.shape[0]
    user_idx = sparse_feature[:, 0].astype(jnp.int32)
    item_idx = sparse_feature[:, 1].astype(jnp.int32)

    a0 = (Wu1 @ Wu2 @ W3[:10]).reshape(-1)
    a1 = (Wi1 @ Wi2 @ W3[10:]).reshape(-1)
    c = ((bu1 @ Wu2 + bu2) @ W3[:10, 0]
         + (bi1 @ Wi2 + bi2) @ W3[10:, 0] + b3[0])
    cv = jnp.full((L,), c, jnp.float32)

    t0 = _folded_dots(E0, a0)
    t1 = _folded_dots(E1, a1)

    iu = user_idx.reshape(-1, CH)
    ii = item_idx.reshape(-1, CH)
    su = iu >> 7
    si = ii >> 7
    lu = iu & (MC - 1)
    li = ii & (MC - 1)

    sc_call = _build_sc_call(B)
    return sc_call(t0, t1, su, si, lu, li, cv)
